# Initial kernel scaffold; baseline (speedup 1.0000x reference)
import jax
import jax.numpy as jnp
from jax.experimental import pallas as pl


def kernel(x, edge_index, W, a_src, a_dst):
    raise NotImplementedError("write your pallas kernel here")


# speedup vs baseline: 3.6177x; 3.6177x over previous
import functools

import jax
import jax.numpy as jnp
from jax import lax
from jax.experimental import pallas as pl
from jax.experimental.pallas import tpu as pltpu
from jax.experimental.pallas import tpu_sc as plsc

N = 10000
E = 160000
D = 256
DW = 272
NPAD = 10240
NC = 2
NS = 16
EB = 64
NBLK = 158
EPT = EB * NBLK
EP = EPT * NS
HALF = N // 2
ACC_ROWS = 5008
RPT = ACC_ROWS // NS
_RCHUNKS = [(o, min(EB, RPT - o)) for o in range(0, RPT, EB)]


def _tc1_body(x_ref, w_ref, a2_ref, h_ref, sd_ref, cmax_ref):
    h = jnp.dot(x_ref[...], w_ref[...], preferred_element_type=jnp.float32)
    h_ref[...] = h
    sdv = jnp.dot(h, a2_ref[...], preferred_element_type=jnp.float32)
    sd_ref[...] = sdv

    @pl.when(pl.program_id(0) == 0)
    def _():
        cmax_ref[...] = jnp.full((2, 128), -jnp.inf, jnp.float32)

    ms = jnp.max(sdv[:, 0])
    md = jnp.max(sdv[:, 1])
    cur = jnp.concatenate(
        [jnp.full((1, 128), ms, jnp.float32), jnp.full((1, 128), md, jnp.float32)], axis=0)
    cmax_ref[...] = jnp.maximum(cmax_ref[...], cur)


def _tc1(x_aug, w_aug, a2):
    blk = 1024
    grid = NPAD // blk
    return pl.pallas_call(
        _tc1_body,
        grid=(grid,),
        in_specs=[
            pl.BlockSpec((blk, DW), lambda i: (i, 0)),
            pl.BlockSpec((DW, DW), lambda i: (0, 0)),
            pl.BlockSpec((DW, 128), lambda i: (0, 0)),
        ],
        out_specs=[
            pl.BlockSpec((blk, DW), lambda i: (i, 0)),
            pl.BlockSpec((blk, 128), lambda i: (i, 0)),
            pl.BlockSpec((2, 128), lambda i: (0, 0)),
        ],
        out_shape=[
            jax.ShapeDtypeStruct((NPAD, DW), jnp.float32),
            jax.ShapeDtypeStruct((NPAD, 128), jnp.float32),
            jax.ShapeDtypeStruct((2, 128), jnp.float32),
        ],
    )(x_aug, w_aug, a2)


def _sc_body(src_hbm, dst_hbm, s_hbm, d_hbm, cmax_hbm, haug_hbm, msg_hbm,
             s_loc, d_loc, src_b, dst_b, alpha_b, ldst_b, rowbuf, cv, acc):
    core = lax.axis_index("c")
    sub = lax.axis_index("s")
    zero16 = jnp.zeros((16,), jnp.float32)

    pltpu.sync_copy(s_hbm, s_loc)
    pltpu.sync_copy(d_hbm, d_loc)
    ebase = sub * EPT
    pltpu.sync_copy(cmax_hbm.at[:, pl.ds(0, 16)], cv)
    tmp = cv[0, :] + cv[1, :]
    cbound = jnp.where(tmp > 0, tmp, 0.2 * tmp)

    def _zrow(r, _):
        for c in range(DW // 16):
            rowbuf[r, pl.ds(16 * c, 16)] = zero16
        return 0
    lax.fori_loop(0, EB, _zrow, 0)
    abase = sub * RPT
    for (o, sz) in _RCHUNKS:
        pltpu.sync_copy(rowbuf.at[pl.ds(0, sz)], acc.at[pl.ds(abase + o, sz)])
    plsc.subcore_barrier()

    dbase = core * HALF

    def _block(b, _):
        pltpu.sync_copy(src_hbm.at[pl.ds(ebase + b * EB, EB)], src_b)
        pltpu.sync_copy(dst_hbm.at[pl.ds(ebase + b * EB, EB)], dst_b)
        for g in range(EB // 16):
            off = g * 16
            s16 = src_b[pl.ds(off, 16)]
            d16 = dst_b[pl.ds(off, 16)]
            sv = plsc.load_gather(s_loc, [s16])
            dv = plsc.load_gather(d_loc, [d16])
            e = sv + dv
            e = jnp.where(e > 0, e, 0.2 * e)
            alpha_b[pl.ds(g * 16, 16)] = jnp.exp(e - cbound)
            ld = d16 - dbase
            inr = (ld >= 0) & (ld < HALF)
            ldst_b[0, pl.ds(g * 16, 16)] = jnp.where(inr, ld, HALF)

        pltpu.sync_copy(haug_hbm.at[src_b], rowbuf)

        def _srow(r, _):
            spl = plsc.load_gather(alpha_b, [jnp.full((16,), r, jnp.int32)])
            for c in range(DW // 16):
                sl = pl.ds(16 * c, 16)
                rowbuf[r, sl] = rowbuf[r, sl] * spl
            return 0
        lax.fori_loop(0, EB, _srow, 0)

        pltpu.sync_copy(rowbuf, acc.at[ldst_b.at[0]], add=True)
        return 0

    lax.fori_loop(0, NBLK, _block, 0)
    plsc.subcore_barrier()

    for (o, sz) in _RCHUNKS:
        pltpu.sync_copy(acc.at[pl.ds(abase + o, sz)],
                        msg_hbm.at[core, pl.ds(abase + o, sz)])


def _sc_call(src_p, dst_p, s1, d1, cmax, haug):
    mesh = plsc.VectorSubcoreMesh(core_axis_name="c", subcore_axis_name="s",
                                  num_cores=NC, num_subcores=NS)
    f = pl.kernel(
        _sc_body,
        out_type=jax.ShapeDtypeStruct((NC, ACC_ROWS, DW), jnp.float32),
        mesh=mesh,
        compiler_params=pltpu.CompilerParams(use_tc_tiling_on_sc=False,
                                             needs_layout_passes=False),
        scratch_types=[
            pltpu.VMEM((NPAD,), jnp.float32),
            pltpu.VMEM((NPAD,), jnp.float32),
            pltpu.VMEM((EB,), jnp.int32),
            pltpu.VMEM((EB,), jnp.int32),
            pltpu.VMEM((EB,), jnp.float32),
            pltpu.VMEM((1, EB), jnp.int32),
            pltpu.VMEM((EB, DW), jnp.float32),
            pltpu.VMEM((2, 16), jnp.float32),
            pltpu.VMEM_SHARED((ACC_ROWS, DW), jnp.float32),
        ],
    )
    return f(src_p, dst_p, s1, d1, cmax, haug)


def _tc2_body(m_ref, o_ref):
    blk = m_ref[0]
    den = blk[:, 256:257]
    o_ref[...] = blk[:, :256] / jnp.clip(den, 1e-9, None)


def _tc2(msg):
    blk = 1000
    return pl.pallas_call(
        _tc2_body,
        grid=(N // blk,),
        in_specs=[
            pl.BlockSpec((1, blk, DW), lambda i: (i // 5, i % 5, 0)),
        ],
        out_specs=pl.BlockSpec((blk, 256), lambda i: (i, 0)),
        out_shape=jax.ShapeDtypeStruct((N, 256), jnp.float32),
    )(msg)


def kernel(x, edge_index, W, a_src, a_dst):
    f32 = jnp.float32
    x_aug = jnp.zeros((NPAD, DW), f32)
    x_aug = x_aug.at[:N, :D].set(x).at[:N, D].set(1.0)
    w_aug = jnp.zeros((DW, DW), f32).at[:D, :D].set(W).at[D, D].set(1.0)
    a2 = jnp.zeros((DW, 128), f32).at[:D, 0].set(a_src).at[:D, 1].set(a_dst)

    haug, sd128, cmax = _tc1(x_aug, w_aug, a2)
    s1 = sd128[:, 0]
    d1 = sd128[:, 1]

    src = edge_index[0]
    dst = edge_index[1]
    src_p = jnp.zeros((EP,), jnp.int32).at[:E].set(src)
    dst_p = jnp.full((EP,), N, jnp.int32).at[:E].set(dst)

    msg = _sc_call(src_p, dst_p, s1, d1, cmax, haug)
    return _tc2(msg)


# speedup vs baseline: 7.2609x; 2.0071x over previous
import jax
import jax.numpy as jnp
from jax import lax
from jax.experimental import pallas as pl
from jax.experimental.pallas import tpu as pltpu
from jax.experimental.pallas import tpu_sc as plsc

N = 10000
E = 160000
D = 256
DW = 272
NPAD = 10240
NC = 2
NS = 16
EPA = 5008
EP = EPA * NC * NS
EB = 80
NBLK = 125
EPT = EB * NBLK
HALF = N // 2
ACC_ROWS = 5008
RPT = ACC_ROWS // NS
_RCHUNKS = [(o, min(EB, RPT - o)) for o in range(0, RPT, EB)]

_SC_PARAMS = pltpu.CompilerParams(use_tc_tiling_on_sc=False,
                                  needs_layout_passes=False)


def _mesh():
    return plsc.VectorSubcoreMesh(core_axis_name="c", subcore_axis_name="s",
                                  num_cores=NC, num_subcores=NS)


def _tc1_body(x_ref, w_ref, a2_ref, h_ref, sd_ref, cmax_ref):
    h = jnp.dot(x_ref[...], w_ref[...], preferred_element_type=jnp.float32)
    h_ref[...] = h
    sdv = jnp.dot(h, a2_ref[...], preferred_element_type=jnp.float32)
    sd_ref[...] = sdv

    @pl.when(pl.program_id(0) == 0)
    def _():
        cmax_ref[...] = jnp.full((2, 128), -jnp.inf, jnp.float32)

    ms = jnp.max(sdv[:, 0])
    md = jnp.max(sdv[:, 1])
    cur = jnp.concatenate(
        [jnp.full((1, 128), ms, jnp.float32), jnp.full((1, 128), md, jnp.float32)], axis=0)
    cmax_ref[...] = jnp.maximum(cmax_ref[...], cur)


def _tc1(x_aug, w_aug, a2):
    blk = 1024
    grid = NPAD // blk
    return pl.pallas_call(
        _tc1_body,
        grid=(grid,),
        in_specs=[
            pl.BlockSpec((blk, DW), lambda i: (i, 0)),
            pl.BlockSpec((DW, DW), lambda i: (0, 0)),
            pl.BlockSpec((DW, 128), lambda i: (0, 0)),
        ],
        out_specs=[
            pl.BlockSpec((blk, DW), lambda i: (i, 0)),
            pl.BlockSpec((blk, 128), lambda i: (i, 0)),
            pl.BlockSpec((2, 128), lambda i: (0, 0)),
        ],
        out_shape=[
            jax.ShapeDtypeStruct((NPAD, DW), jnp.float32),
            jax.ShapeDtypeStruct((NPAD, 128), jnp.float32),
            jax.ShapeDtypeStruct((2, 128), jnp.float32),
        ],
    )(x_aug, w_aug, a2)


def _sca_body(src_hbm, dst_hbm, s_hbm, d_hbm, cmax_hbm, rec_hbm,
              s_loc, d_loc, src_t, dst_t, rec_t, cv):
    core = lax.axis_index("c")
    sub = lax.axis_index("s")
    wid = sub * NC + core
    ebase = wid * EPA

    pltpu.sync_copy(s_hbm, s_loc)
    pltpu.sync_copy(d_hbm, d_loc)
    pltpu.sync_copy(src_hbm.at[pl.ds(ebase, EPA)], src_t)
    pltpu.sync_copy(dst_hbm.at[pl.ds(ebase, EPA)], dst_t)
    pltpu.sync_copy(cmax_hbm.at[:, pl.ds(0, 16)], cv)
    tmp = cv[0, :] + cv[1, :]
    cbound = jnp.where(tmp > 0, tmp, 0.2 * tmp)

    iota = lax.iota(jnp.int32, 16)

    def _grp(g, _):
        off = g * 16
        s16 = src_t[pl.ds(off, 16)]
        d16 = dst_t[pl.ds(off, 16)]
        sv = plsc.load_gather(s_loc, [s16])
        dv = plsc.load_gather(d_loc, [d16])
        e = sv + dv
        e = jnp.where(e > 0, e, 0.2 * e)
        alpha = jnp.exp(e - cbound)
        ki4 = (off + iota) * 4
        plsc.store_scatter(rec_t, [ki4], s16)
        plsc.store_scatter(rec_t, [ki4 + 1], d16)
        plsc.store_scatter(rec_t, [ki4 + 2], plsc.bitcast(alpha, jnp.int32))
        return 0

    lax.fori_loop(0, EPA // 16, _grp, 0)
    pltpu.sync_copy(rec_t, rec_hbm.at[pl.ds(ebase * 4, EPA * 4)])


def _sca_call(src_p, dst_p, s1, d1, cmax):
    f = pl.kernel(
        _sca_body,
        out_type=jax.ShapeDtypeStruct((EP * 4,), jnp.int32),
        mesh=_mesh(),
        compiler_params=_SC_PARAMS,
        scratch_types=[
            pltpu.VMEM((NPAD,), jnp.float32),
            pltpu.VMEM((NPAD,), jnp.float32),
            pltpu.VMEM((EPA,), jnp.int32),
            pltpu.VMEM((EPA,), jnp.int32),
            pltpu.VMEM((EPA * 4,), jnp.int32),
            pltpu.VMEM((2, 16), jnp.float32),
        ],
    )
    return f(src_p, dst_p, s1, d1, cmax)


def _scb_body(rec_hbm, haug_hbm, msg_hbm,
              rec, src_b, alpha_b, ldst_b, rowbuf, se, sg, ss, acc):
    core = lax.axis_index("c")
    sub = lax.axis_index("s")
    zero16 = jnp.zeros((16,), jnp.float32)
    iota = lax.iota(jnp.int32, 16)
    ebase = sub * EPT
    dbase = core * HALF

    def _zrow(r, _):
        for c in range(DW // 16):
            rowbuf[0, r, pl.ds(16 * c, 16)] = zero16
        return 0
    lax.fori_loop(0, EB, _zrow, 0)
    abase = sub * RPT
    for (o, sz) in _RCHUNKS:
        pltpu.sync_copy(rowbuf.at[0, pl.ds(0, sz)], acc.at[pl.ds(abase + o, sz)])
    plsc.subcore_barrier()

    def _stage(j, t, sem):
        pltpu.async_copy(
            rec_hbm.at[pl.ds((ebase + j * EB) * 4, EB * 4)], rec.at[t], sem)

    def _extract(t):
        for g in range(EB // 16):
            ki4 = (g * 16 + iota) * 4
            sl = pl.ds(g * 16, 16)
            src_b[t, sl] = plsc.load_gather(rec.at[t], [ki4])
            alpha_b[t, sl] = plsc.bitcast(
                plsc.load_gather(rec.at[t], [ki4 + 2]), jnp.float32)
            ld = plsc.load_gather(rec.at[t], [ki4 + 1]) - dbase
            inr = (ld >= 0) & (ld < HALF)
            ldst_b[t, sl] = jnp.where(inr, ld, HALF)

    _stage(0, 0, se.at[0])
    pltpu.make_async_copy(
        rec_hbm.at[pl.ds(ebase * 4, EB * 4)], rec.at[0], se.at[0]).wait()
    _extract(0)
    pltpu.async_copy(haug_hbm.at[src_b.at[0]], rowbuf.at[0], sg.at[0])
    _stage(1, 1, se.at[1])

    def _block(i, _):
        p = lax.rem(i, 2)
        q = 1 - p

        @pl.when(i + 1 < NBLK)
        def _():
            pltpu.make_async_copy(
                rec_hbm.at[pl.ds(ebase * 4, EB * 4)], rec.at[q], se.at[q]).wait()

            @pl.when(i >= 1)
            def _():
                pltpu.make_async_copy(
                    haug_hbm.at[pl.ds(0, EB)], rowbuf.at[q], ss.at[q]).wait()
            _extract(q)
            pltpu.async_copy(haug_hbm.at[src_b.at[q]], rowbuf.at[q], sg.at[q])

            @pl.when(i + 2 < NBLK)
            def _():
                _stage(i + 2, p, se.at[p])

        pltpu.make_async_copy(
            haug_hbm.at[src_b.at[p]], rowbuf.at[p], sg.at[p]).wait()

        def _srow(r, _):
            spl = plsc.load_gather(alpha_b.at[p], [jnp.full((16,), r, jnp.int32)])
            for c in range(DW // 16):
                sl = pl.ds(16 * c, 16)
                rowbuf[p, r, sl] = rowbuf[p, r, sl] * spl
            return 0
        lax.fori_loop(0, EB, _srow, 0)
        pltpu.async_copy(rowbuf.at[p], acc.at[ldst_b.at[p]], ss.at[p], add=True)
        return 0

    lax.fori_loop(0, NBLK, _block, 0)
    pltpu.make_async_copy(haug_hbm.at[pl.ds(0, EB)], rowbuf.at[0], ss.at[0]).wait()
    pltpu.make_async_copy(haug_hbm.at[pl.ds(0, EB)], rowbuf.at[1], ss.at[1]).wait()
    plsc.subcore_barrier()

    for (o, sz) in _RCHUNKS:
        pltpu.sync_copy(acc.at[pl.ds(abase + o, sz)],
                        msg_hbm.at[core, pl.ds(abase + o, sz)])


def _scb_call(rec, haug):
    f = pl.kernel(
        _scb_body,
        out_type=jax.ShapeDtypeStruct((NC, ACC_ROWS, DW), jnp.float32),
        mesh=_mesh(),
        compiler_params=_SC_PARAMS,
        scratch_types=[
            pltpu.VMEM((2, EB * 4), jnp.int32),
            pltpu.VMEM((2, EB), jnp.int32),
            pltpu.VMEM((2, EB), jnp.float32),
            pltpu.VMEM((2, EB), jnp.int32),
            pltpu.VMEM((2, EB, DW), jnp.float32),
            pltpu.SemaphoreType.DMA((2,)),
            pltpu.SemaphoreType.DMA((2,)),
            pltpu.SemaphoreType.DMA((2,)),
            pltpu.VMEM_SHARED((ACC_ROWS, DW), jnp.float32),
        ],
    )
    return f(rec, haug)


def _tc2_body(m_ref, o_ref):
    blk = m_ref[0]
    den = blk[:, 256:257]
    o_ref[...] = blk[:, :256] / jnp.clip(den, 1e-9, None)


def _tc2(msg):
    blk = 1000
    return pl.pallas_call(
        _tc2_body,
        grid=(N // blk,),
        in_specs=[
            pl.BlockSpec((1, blk, DW), lambda i: (i // 5, i % 5, 0)),
        ],
        out_specs=pl.BlockSpec((blk, 256), lambda i: (i, 0)),
        out_shape=jax.ShapeDtypeStruct((N, 256), jnp.float32),
    )(msg)


def kernel(x, edge_index, W, a_src, a_dst):
    f32 = jnp.float32
    x_aug = jnp.zeros((NPAD, DW), f32)
    x_aug = x_aug.at[:N, :D].set(x).at[:N, D].set(1.0)
    w_aug = jnp.zeros((DW, DW), f32).at[:D, :D].set(W).at[D, D].set(1.0)
    a2 = jnp.zeros((DW, 128), f32).at[:D, 0].set(a_src).at[:D, 1].set(a_dst)

    haug, sd128, cmax = _tc1(x_aug, w_aug, a2)
    s1 = sd128[:, 0]
    d1 = sd128[:, 1]

    src = edge_index[0]
    dst = edge_index[1]
    src_p = jnp.zeros((EP,), jnp.int32).at[:E].set(src)
    dst_p = jnp.full((EP,), N, jnp.int32).at[:E].set(dst)

    rec = _sca_call(src_p, dst_p, s1, d1, cmax)
    msg = _scb_call(rec, haug)
    return _tc2(msg)
